# baseline (speedup 1.0000x reference)
import jax
import jax.numpy as jnp
from jax import lax
from jax.experimental import pallas as pl
from jax.experimental.pallas import tpu as pltpu
from jax.experimental.pallas import tpu_sc as plsc

VOCAB = 1000000
D = 32
B = 16384
L = 50
LP = 64

NC = 2
NS = 16
NW = NC * NS
BW = B // NW
C = 64
CHUNKS = BW // C
CL = C * L


def _sc_body(user_id_hbm, hist_hbm, mask_hbm, item_id_hbm,
             user_table_hbm, item_table_hbm, out_hbm,
             uidx_v, iidx_v, ub_v, ib_v, hidx0_v, hidx1_v,
             mask0_v, mask1_v, rows_v, out_v,
             sem_g, sem_p0, sem_p1):
    wid = lax.axis_index("s") * NC + lax.axis_index("c")
    base = pl.multiple_of(wid * BW, BW)
    hidx_b = (hidx0_v, hidx1_v)
    mask_b = (mask0_v, mask1_v)
    sem_p = (sem_p0, sem_p1)

    pltpu.sync_copy(user_id_hbm.at[pl.ds(base, BW)], uidx_v)
    pltpu.sync_copy(item_id_hbm.at[pl.ds(base, BW)], iidx_v)

    def idx_copies(c, buf):
        r0 = pl.multiple_of(base * L, BW * L) + c * CL
        m0 = pl.multiple_of(base * LP, BW * LP) + c * (C * LP)
        return (pltpu.make_async_copy(hist_hbm.at[pl.ds(r0, CL)],
                                      hidx_b[buf], sem_p[buf]),
                pltpu.make_async_copy(mask_hbm.at[pl.ds(m0, C * LP)],
                                      mask_b[buf], sem_p[buf]))

    for cp in idx_copies(0, 0):
        cp.start()

    for c in range(CHUNKS):
        buf = c % 2
        for cp in idx_copies(c, buf):
            cp.wait()
        gathers = [
            pltpu.async_copy(item_table_hbm.at[hidx_b[buf]], rows_v, sem_g),
            pltpu.async_copy(
                user_table_hbm.at[uidx_v.at[pl.ds(c * C, C)]],
                ub_v, sem_g),
            pltpu.async_copy(
                item_table_hbm.at[iidx_v.at[pl.ds(c * C, C)]],
                ib_v, sem_g),
        ]
        if c + 1 < CHUNKS:
            for cp in idx_copies(c + 1, 1 - buf):
                cp.start()
        for cp in gathers:
            cp.wait()

        mask_v = mask_b[buf]

        def b_body(b, _):
            r0 = b * L
            mv = [mask_v[pl.ds(b * LP + 16 * k, 16)] for k in range(4)]
            acc = [jnp.zeros((16,), jnp.float32) for _ in range(8)]
            for l in range(L):
                m = mv[l // 16][l % 16]
                k = l % 4
                acc[2 * k] += m * rows_v[r0 + l, pl.ds(0, 16)]
                acc[2 * k + 1] += m * rows_v[r0 + l, pl.ds(16, 16)]
            scale = jnp.float32(1.0 / L)
            lo = (acc[0] + acc[2]) + (acc[4] + acc[6])
            hi = (acc[1] + acc[3]) + (acc[5] + acc[7])
            out_v[b, pl.ds(0, 16)] = ub_v[b, pl.ds(0, 16)]
            out_v[b, pl.ds(16, 16)] = ub_v[b, pl.ds(16, 16)]
            out_v[b, pl.ds(D, 16)] = lo * scale
            out_v[b, pl.ds(D + 16, 16)] = hi * scale
            out_v[b, pl.ds(2 * D, 16)] = ib_v[b, pl.ds(0, 16)]
            out_v[b, pl.ds(2 * D + 16, 16)] = ib_v[b, pl.ds(16, 16)]
            return 0

        lax.fori_loop(0, C, b_body, 0)
        pltpu.sync_copy(out_v, out_hbm.at[pl.ds(base + c * C, C), :])


@jax.jit
def _sc_call(user_id, hist_flat, mask_flat, item_id, user_table, item_table):
    mesh = plsc.VectorSubcoreMesh(core_axis_name="c", subcore_axis_name="s",
                                  num_cores=NC, num_subcores=NS)
    return pl.kernel(
        _sc_body,
        out_type=jax.ShapeDtypeStruct((B, 3 * D), jnp.float32),
        mesh=mesh,
        compiler_params=pltpu.CompilerParams(use_tc_tiling_on_sc=False),
        scratch_types=[
            pltpu.VMEM((BW,), jnp.int32),
            pltpu.VMEM((BW,), jnp.int32),
            pltpu.VMEM((C, D), jnp.float32),
            pltpu.VMEM((C, D), jnp.float32),
            pltpu.VMEM((CL,), jnp.int32),
            pltpu.VMEM((CL,), jnp.int32),
            pltpu.VMEM((C * LP,), jnp.float32),
            pltpu.VMEM((C * LP,), jnp.float32),
            pltpu.VMEM((CL, D), jnp.float32),
            pltpu.VMEM((C, 3 * D), jnp.float32),
            pltpu.SemaphoreType.DMA,
            pltpu.SemaphoreType.DMA,
            pltpu.SemaphoreType.DMA,
        ],
    )(user_id, hist_flat, mask_flat, item_id, user_table, item_table)


def kernel(user_id, user_hist, hist_mask, item_id, user_table, item_table):
    user_id = user_id.astype(jnp.int32)
    item_id = item_id.astype(jnp.int32)
    hist_flat = user_hist.astype(jnp.int32).reshape(-1)
    mask_flat = jnp.pad(hist_mask, ((0, 0), (0, LP - L))).reshape(-1)
    return _sc_call(user_id, hist_flat, mask_flat, item_id,
                    user_table, item_table)


# speedup vs baseline: 1.0297x; 1.0297x over previous
import jax
import jax.numpy as jnp
from jax import lax
from jax.experimental import pallas as pl
from jax.experimental.pallas import tpu as pltpu
from jax.experimental.pallas import tpu_sc as plsc

VOCAB = 1000000
D = 32
B = 16384
L = 50
LP = 64

NC = 2
NS = 16
NW = NC * NS
BW = B // NW
C = 32
CHUNKS = BW // C
CL = C * L
CLP = C * LP


def _sc_body(user_id_hbm, hist_hbm, mask_hbm, item_id_hbm,
             user_table_hbm, item_table_hbm, out_hbm,
             uidx_v, iidx_v, ub0_v, ub1_v, ib0_v, ib1_v,
             hidx0_v, hidx1_v, mask0_v, mask1_v,
             rows0_v, rows1_v, out0_v, out1_v,
             sem_g0, sem_g1, sem_p0, sem_p1, sem_o0, sem_o1):
    wid = lax.axis_index("s") * NC + lax.axis_index("c")
    base = pl.multiple_of(wid * BW, BW)
    ub_b = (ub0_v, ub1_v)
    ib_b = (ib0_v, ib1_v)
    hidx_b = (hidx0_v, hidx1_v)
    mask_b = (mask0_v, mask1_v)
    rows_b = (rows0_v, rows1_v)
    out_b = (out0_v, out1_v)
    sem_g = (sem_g0, sem_g1)
    sem_p = (sem_p0, sem_p1)
    sem_o = (sem_o0, sem_o1)

    pltpu.sync_copy(user_id_hbm.at[pl.ds(base, BW)], uidx_v)
    pltpu.sync_copy(item_id_hbm.at[pl.ds(base, BW)], iidx_v)

    def idx_copies(t, p):
        r0 = pl.multiple_of(base * L, BW * L) + t * CL
        m0 = pl.multiple_of(base * LP, BW * LP) + t * CLP
        return (pltpu.make_async_copy(hist_hbm.at[pl.ds(r0, CL)],
                                      hidx_b[p], sem_p[p]),
                pltpu.make_async_copy(mask_hbm.at[pl.ds(m0, CLP)],
                                      mask_b[p], sem_p[p]))

    def gathers(t, p):
        return (pltpu.make_async_copy(item_table_hbm.at[hidx_b[p]],
                                      rows_b[p], sem_g[p]),
                pltpu.make_async_copy(
                    user_table_hbm.at[uidx_v.at[pl.ds(t * C, C)]],
                    ub_b[p], sem_g[p]),
                pltpu.make_async_copy(
                    item_table_hbm.at[iidx_v.at[pl.ds(t * C, C)]],
                    ib_b[p], sem_g[p]))

    def out_copy(t, p):
        return pltpu.make_async_copy(
            out_b[p], out_hbm.at[pl.ds(base + t * C, C), :], sem_o[p])

    for cp in idx_copies(0, 0):
        cp.start()
    for cp in idx_copies(0, 0):
        cp.wait()
    for cp in gathers(0, 0):
        cp.start()
    for cp in idx_copies(1, 1):
        cp.start()

    def compute_chunk(t, p):
        rows_v, mask_v, out_v = rows_b[p], mask_b[p], out_b[p]
        ub_v, ib_v = ub_b[p], ib_b[p]

        def b_body(b, _):
            r0 = b * L
            mv = [mask_v[pl.ds(b * LP + 16 * k, 16)] for k in range(4)]
            acc = [jnp.zeros((16,), jnp.float32) for _ in range(8)]
            for l in range(L):
                m = mv[l // 16][l % 16]
                k = l % 4
                acc[2 * k] += m * rows_v[r0 + l, pl.ds(0, 16)]
                acc[2 * k + 1] += m * rows_v[r0 + l, pl.ds(16, 16)]
            scale = jnp.float32(1.0 / L)
            lo = (acc[0] + acc[2]) + (acc[4] + acc[6])
            hi = (acc[1] + acc[3]) + (acc[5] + acc[7])
            out_v[b, pl.ds(0, 16)] = ub_v[b, pl.ds(0, 16)]
            out_v[b, pl.ds(16, 16)] = ub_v[b, pl.ds(16, 16)]
            out_v[b, pl.ds(D, 16)] = lo * scale
            out_v[b, pl.ds(D + 16, 16)] = hi * scale
            out_v[b, pl.ds(2 * D, 16)] = ib_v[b, pl.ds(0, 16)]
            out_v[b, pl.ds(2 * D + 16, 16)] = ib_v[b, pl.ds(16, 16)]
            return 0

        lax.fori_loop(0, C, b_body, 0)

    def loop_body(tt, _):
        for p in range(2):
            t = tt * 2 + p
            q = 1 - p
            for cp in gathers(t, p):
                cp.wait()

            @pl.when(t + 1 < CHUNKS)
            def _():
                for cp in idx_copies(t + 1, q):
                    cp.wait()
                for cp in gathers(t + 1, q):
                    cp.start()

            @pl.when(t >= 2)
            def _():
                out_copy(t - 2, p).wait()

            compute_chunk(t, p)
            out_copy(t, p).start()

            @pl.when(t + 2 < CHUNKS)
            def _():
                for cp in idx_copies(t + 2, p):
                    cp.start()
        return 0

    lax.fori_loop(0, CHUNKS // 2, loop_body, 0)
    out_copy(CHUNKS - 2, 0).wait()
    out_copy(CHUNKS - 1, 1).wait()


@jax.jit
def _sc_call(user_id, hist_flat, mask_flat, item_id, user_table, item_table):
    mesh = plsc.VectorSubcoreMesh(core_axis_name="c", subcore_axis_name="s",
                                  num_cores=NC, num_subcores=NS)
    return pl.kernel(
        _sc_body,
        out_type=jax.ShapeDtypeStruct((B, 3 * D), jnp.float32),
        mesh=mesh,
        compiler_params=pltpu.CompilerParams(use_tc_tiling_on_sc=False),
        scratch_types=[
            pltpu.VMEM((BW,), jnp.int32),
            pltpu.VMEM((BW,), jnp.int32),
            pltpu.VMEM((C, D), jnp.float32),
            pltpu.VMEM((C, D), jnp.float32),
            pltpu.VMEM((C, D), jnp.float32),
            pltpu.VMEM((C, D), jnp.float32),
            pltpu.VMEM((CL,), jnp.int32),
            pltpu.VMEM((CL,), jnp.int32),
            pltpu.VMEM((CLP,), jnp.float32),
            pltpu.VMEM((CLP,), jnp.float32),
            pltpu.VMEM((CL, D), jnp.float32),
            pltpu.VMEM((CL, D), jnp.float32),
            pltpu.VMEM((C, 3 * D), jnp.float32),
            pltpu.VMEM((C, 3 * D), jnp.float32),
            pltpu.SemaphoreType.DMA,
            pltpu.SemaphoreType.DMA,
            pltpu.SemaphoreType.DMA,
            pltpu.SemaphoreType.DMA,
            pltpu.SemaphoreType.DMA,
            pltpu.SemaphoreType.DMA,
        ],
    )(user_id, hist_flat, mask_flat, item_id, user_table, item_table)


def kernel(user_id, user_hist, hist_mask, item_id, user_table, item_table):
    user_id = user_id.astype(jnp.int32)
    item_id = item_id.astype(jnp.int32)
    hist_flat = user_hist.astype(jnp.int32).reshape(-1)
    mask_flat = jnp.pad(hist_mask, ((0, 0), (0, LP - L))).reshape(-1)
    return _sc_call(user_id, hist_flat, mask_flat, item_id,
                    user_table, item_table)
